# Initial kernel scaffold; baseline (speedup 1.0000x reference)
import jax
import jax.numpy as jnp
from jax.experimental import pallas as pl


def kernel(unknown, known, unknow_feats, known_feats, W, b):
    raise NotImplementedError("write your pallas kernel here")


# speedup vs baseline: 31.7963x; 31.7963x over previous
import jax
import jax.numpy as jnp
from jax.experimental import pallas as pl
from jax.experimental.pallas import tpu as pltpu

B, N, M, C1, C2, CO = 4, 8192, 1024, 32, 64, 128
NB = 512


def _fp_body(known_ref, unknown_t_ref, uf_ref, kf_ref, w_ref, b_ref, out_ref):
    kx = known_ref[:, 0:1]
    ky = known_ref[:, 1:2]
    kz = known_ref[:, 2:3]
    ux = unknown_t_ref[0:1, :]
    uy = unknown_t_ref[1:2, :]
    uz = unknown_t_ref[2:3, :]

    dx = ux - kx
    dy = uy - ky
    dz = uz - kz
    d2 = dx * dx + dy * dy + dz * dz

    sub_iota = jax.lax.broadcasted_iota(jnp.int32, (M, NB), 0)
    big = jnp.float32(jnp.inf)

    def min3(d):
        m1 = jnp.min(d, axis=0, keepdims=True)
        i1 = jnp.min(jnp.where(d == m1, sub_iota, M), axis=0, keepdims=True)
        d = jnp.where(sub_iota == i1, big, d)
        m2 = jnp.min(d, axis=0, keepdims=True)
        i2 = jnp.min(jnp.where(d == m2, sub_iota, M), axis=0, keepdims=True)
        d = jnp.where(sub_iota == i2, big, d)
        m3 = jnp.min(d, axis=0, keepdims=True)
        i3 = jnp.min(jnp.where(d == m3, sub_iota, M), axis=0, keepdims=True)
        return (m1, m2, m3), (i1, i2, i3)

    (m1, m2, m3), (i1, i2, i3) = min3(d2)

    def recip(m):
        return 1.0 / (jnp.sqrt(jnp.maximum(m, 0.0)) + 1e-8)

    r1, r2, r3 = recip(m1), recip(m2), recip(m3)
    norm = r1 + r2 + r3
    w1, w2, w3 = r1 / norm, r2 / norm, r3 / norm

    zero = jnp.float32(0.0)
    comb = (
        jnp.where(sub_iota == i1, w1, zero)
        + jnp.where(sub_iota == i2, w2, zero)
        + jnp.where(sub_iota == i3, w3, zero)
    )

    interp = jnp.dot(kf_ref[...], comb, preferred_element_type=jnp.float32)

    w2m = w_ref[:, 0:C2]
    w1m = w_ref[:, C2:C2 + C1]
    out = (
        jnp.dot(w2m, interp, preferred_element_type=jnp.float32)
        + jnp.dot(w1m, uf_ref[...], preferred_element_type=jnp.float32)
        + b_ref[...]
    )
    out_ref[...] = jnp.maximum(out, 0.0)


def kernel(unknown, known, unknow_feats, known_feats, W, b):
    unknown_t = jnp.transpose(unknown, (0, 2, 1))
    b2 = b.reshape(CO, 1)

    grid = (B, N // NB)
    out = pl.pallas_call(
        _fp_body,
        grid=grid,
        in_specs=[
            pl.BlockSpec((None, M, 3), lambda bb, nn: (bb, 0, 0)),
            pl.BlockSpec((None, 3, NB), lambda bb, nn: (bb, 0, nn)),
            pl.BlockSpec((None, C1, NB), lambda bb, nn: (bb, 0, nn)),
            pl.BlockSpec((None, C2, M), lambda bb, nn: (bb, 0, 0)),
            pl.BlockSpec((CO, C1 + C2), lambda bb, nn: (0, 0)),
            pl.BlockSpec((CO, 1), lambda bb, nn: (0, 0)),
        ],
        out_specs=pl.BlockSpec((None, CO, NB), lambda bb, nn: (bb, 0, nn)),
        out_shape=jax.ShapeDtypeStruct((B, CO, N), jnp.float32),
        compiler_params=pltpu.CompilerParams(
            dimension_semantics=("parallel", "parallel"),
        ),
    )(known, unknown_t, unknow_feats, known_feats, W, b2)
    return out
